# Initial kernel scaffold; baseline (speedup 1.0000x reference)
import functools

import jax
import jax.numpy as jnp
from jax import lax
from jax.experimental import pallas as pl
from jax.experimental.pallas import tpu as pltpu
from jax.experimental.pallas import tpu_sc as plsc

VOCAB = 100000
D = 64
BATCH = 1024
STATES = 20
ATOMS = 20
PER_ROW = ATOMS * 3
ROWS = BATCH * (1 + STATES)
NC = 2
NS = 16
NW = NC * NS
ROWS_PER_W = ROWS // NW
R_BLK = 2
IDX_BLK = R_BLK * PER_ROW
N_BLK = ROWS_PER_W // R_BLK
LANES = 16


@functools.partial(
    pl.kernel,
    mesh=plsc.VectorSubcoreMesh(core_axis_name="c", subcore_axis_name="s"),
    out_type=jax.ShapeDtypeStruct((ROWS * D,), jnp.float32),
    scratch_types=[
        pltpu.VMEM((IDX_BLK,), jnp.int32),
        pltpu.VMEM((IDX_BLK, D), jnp.float32),
        pltpu.VMEM((R_BLK * D,), jnp.float32),
        pltpu.SemaphoreType.DMA,
    ],
)
def _pooled_lookup(idx_hbm, table_hbm, out_hbm, idx_v, rows_v, out_v, sem):
    wid = lax.axis_index("s") * NC + lax.axis_index("c")
    row_base = wid * ROWS_PER_W

    def body(i, carry):
        row0 = row_base + i * R_BLK
        pltpu.sync_copy(idx_hbm.at[pl.ds(row0 * PER_ROW, IDX_BLK)], idx_v)
        pltpu.async_copy(table_hbm.at[idx_v], rows_v, sem).wait()
        for r in range(R_BLK):
            for c in range(D // LANES):
                acc = rows_v[r * PER_ROW, pl.ds(c * LANES, LANES)]
                for j in range(1, PER_ROW):
                    acc = acc + rows_v[r * PER_ROW + j, pl.ds(c * LANES, LANES)]
                out_v[pl.ds(r * D + c * LANES, LANES)] = acc * (1.0 / ATOMS)
        pltpu.sync_copy(out_v, out_hbm.at[pl.ds(row0 * D, R_BLK * D)])
        return carry

    lax.fori_loop(0, N_BLK, body, 0)


def kernel(sub_index, derived_sub_indices, action_mask, table):
    obs_idx = sub_index.reshape(BATCH, PER_ROW)
    act_idx = derived_sub_indices.reshape(BATCH * STATES, PER_ROW)
    flat_idx = jnp.concatenate([obs_idx, act_idx], axis=0).reshape(-1)
    out = _pooled_lookup(flat_idx, table).reshape(ROWS, D)
    obs = out[:BATCH]
    act = out[BATCH:].reshape(BATCH, STATES, D)
    return (obs, act, action_mask)


# speedup vs baseline: 7.4539x; 7.4539x over previous
import functools

import jax
import jax.numpy as jnp
from jax import lax
from jax.experimental import pallas as pl
from jax.experimental.pallas import tpu as pltpu
from jax.experimental.pallas import tpu_sc as plsc

VOCAB = 100000
D = 64
BATCH = 1024
STATES = 20
ATOMS = 20
PER_ROW = ATOMS * 3
ROWS = BATCH * (1 + STATES)
NC = 2
NS = 16
NW = NC * NS
ROWS_PER_W = ROWS // NW
R_BLK = 2
IDX_BLK = R_BLK * PER_ROW
N_BLK = ROWS_PER_W // R_BLK
LANES = 16


@functools.partial(
    pl.kernel,
    mesh=plsc.VectorSubcoreMesh(core_axis_name="c", subcore_axis_name="s"),
    out_type=jax.ShapeDtypeStruct((ROWS * D,), jnp.float32),
    compiler_params=pltpu.CompilerParams(use_tc_tiling_on_sc=False),
    scratch_types=[
        pltpu.VMEM((IDX_BLK,), jnp.int32),
        pltpu.VMEM((IDX_BLK, D), jnp.float32),
        pltpu.VMEM((R_BLK * D,), jnp.float32),
        pltpu.SemaphoreType.DMA,
    ],
)
def _pooled_lookup(idx_hbm, table_hbm, out_hbm, idx_v, rows_v, out_v, sem):
    wid = lax.axis_index("s") * NC + lax.axis_index("c")
    row_base = wid * ROWS_PER_W

    def body(i, carry):
        row0 = row_base + i * R_BLK
        pltpu.sync_copy(idx_hbm.at[pl.ds(row0 * PER_ROW, IDX_BLK)], idx_v)
        pltpu.async_copy(table_hbm.at[idx_v], rows_v, sem).wait()
        for r in range(R_BLK):
            for c in range(D // LANES):
                acc = rows_v[r * PER_ROW, pl.ds(c * LANES, LANES)]
                for j in range(1, PER_ROW):
                    acc = acc + rows_v[r * PER_ROW + j, pl.ds(c * LANES, LANES)]
                out_v[pl.ds(r * D + c * LANES, LANES)] = acc * (1.0 / ATOMS)
        pltpu.sync_copy(out_v, out_hbm.at[pl.ds(row0 * D, R_BLK * D)])
        return carry

    lax.fori_loop(0, N_BLK, body, 0)


def kernel(sub_index, derived_sub_indices, action_mask, table):
    obs_idx = sub_index.reshape(BATCH, PER_ROW)
    act_idx = derived_sub_indices.reshape(BATCH * STATES, PER_ROW)
    flat_idx = jnp.concatenate([obs_idx, act_idx], axis=0).reshape(-1)
    out = _pooled_lookup(flat_idx, table).reshape(ROWS, D)
    obs = out[:BATCH]
    act = out[BATCH:].reshape(BATCH, STATES, D)
    return (obs, act, action_mask)


# speedup vs baseline: 14.6826x; 1.9698x over previous
import functools

import jax
import jax.numpy as jnp
from jax import lax
from jax.experimental import pallas as pl
from jax.experimental.pallas import tpu as pltpu
from jax.experimental.pallas import tpu_sc as plsc

VOCAB = 100000
D = 64
BATCH = 1024
STATES = 20
ATOMS = 20
PER_ROW = ATOMS * 3
ROWS = BATCH * (1 + STATES)
NC = 2
NS = 16
NW = NC * NS
ROWS_PER_W = ROWS // NW
R_BLK = 2
IDX_BLK = R_BLK * PER_ROW
N_BLK = ROWS_PER_W // R_BLK
LANES = 16
NCH = D // LANES
J_GRP = 15
SCALE = 1.0 / ATOMS


@functools.partial(
    pl.kernel,
    mesh=plsc.VectorSubcoreMesh(core_axis_name="c", subcore_axis_name="s"),
    out_type=jax.ShapeDtypeStruct((ROWS * D,), jnp.float32),
    compiler_params=pltpu.CompilerParams(use_tc_tiling_on_sc=False),
    scratch_types=[
        pltpu.VMEM((ROWS_PER_W * PER_ROW,), jnp.int32),
        pltpu.VMEM((IDX_BLK, D), jnp.float32),
        pltpu.VMEM((IDX_BLK, D), jnp.float32),
        pltpu.VMEM((R_BLK * D,), jnp.float32),
        pltpu.VMEM((R_BLK * D,), jnp.float32),
        pltpu.SemaphoreType.DMA,
        pltpu.SemaphoreType.DMA,
        pltpu.SemaphoreType.DMA,
        pltpu.SemaphoreType.DMA,
    ],
)
def _pooled_lookup(idx_hbm, table_hbm, out_hbm, idx_all, rows0, rows1,
                   outb0, outb1, semg0, semg1, semo0, semo1):
    wid = lax.axis_index("s") * NC + lax.axis_index("c")
    row_base = wid * ROWS_PER_W

    pltpu.sync_copy(
        idx_hbm.at[pl.ds(row_base * PER_ROW, ROWS_PER_W * PER_ROW)], idx_all)

    def gather(i, rows_b, sem_b):
        return pltpu.make_async_copy(
            table_hbm.at[idx_all.at[pl.ds(i * IDX_BLK, IDX_BLK)]],
            rows_b, sem_b)

    def out_store(i, out_b, sem_b):
        return pltpu.make_async_copy(
            out_b, out_hbm.at[pl.ds((row_base + i * R_BLK) * D, R_BLK * D)],
            sem_b)

    gather(0, rows0, semg0).start()
    gather(1, rows1, semg1).start()

    zeros = jnp.zeros((LANES,), jnp.float32)

    def body(p, carry):
        bufs = ((rows0, semg0, outb0, semo0), (rows1, semg1, outb1, semo1))
        for b, (rows_b, semg_b, out_b, semo_b) in enumerate(bufs):
            i = 2 * p + b
            gather(i, rows_b, semg_b).wait()

            def jbody(jj, accs):
                accs = list(accs)
                for u in range(J_GRP):
                    for r in range(R_BLK):
                        row = r * PER_ROW + jj * J_GRP + u
                        for c in range(NCH):
                            accs[r * NCH + c] = accs[r * NCH + c] + (
                                rows_b[row, pl.ds(c * LANES, LANES)])
                return tuple(accs)

            accs = lax.fori_loop(0, PER_ROW // J_GRP, jbody,
                                 (zeros,) * (R_BLK * NCH))

            @pl.when(i + 2 < N_BLK)
            def _():
                gather(i + 2, rows_b, semg_b).start()

            @pl.when(i >= 2)
            def _():
                out_store(i, out_b, semo_b).wait()

            for r in range(R_BLK):
                for c in range(NCH):
                    out_b[pl.ds(r * D + c * LANES, LANES)] = (
                        accs[r * NCH + c] * SCALE)
            out_store(i, out_b, semo_b).start()
        return carry

    lax.fori_loop(0, N_BLK // 2, body, 0)
    out_store(N_BLK - 2, outb0, semo0).wait()
    out_store(N_BLK - 1, outb1, semo1).wait()


def kernel(sub_index, derived_sub_indices, action_mask, table):
    obs_idx = sub_index.reshape(BATCH, PER_ROW)
    act_idx = derived_sub_indices.reshape(BATCH * STATES, PER_ROW)
    flat_idx = jnp.concatenate([obs_idx, act_idx], axis=0).reshape(-1)
    out = _pooled_lookup(flat_idx, table).reshape(ROWS, D)
    obs = out[:BATCH]
    act = out[BATCH:].reshape(BATCH, STATES, D)
    return (obs, act, action_mask)


# speedup vs baseline: 18.9416x; 1.2901x over previous
import functools

import jax
import jax.numpy as jnp
from jax import lax
from jax.experimental import pallas as pl
from jax.experimental.pallas import tpu as pltpu
from jax.experimental.pallas import tpu_sc as plsc

VOCAB = 100000
D = 64
BATCH = 1024
STATES = 20
ATOMS = 20
PER_ROW = ATOMS * 3
ROWS = BATCH * (1 + STATES)
NC = 2
NS = 16
NW = NC * NS
ROWS_PER_W = ROWS // NW
R_BLK = 2
IDX_BLK = R_BLK * PER_ROW
N_BLK = ROWS_PER_W // R_BLK
NBUF = 4
LANES = 16
NCH = D // LANES
J_GRP = 15
SCALE = 1.0 / ATOMS


@functools.partial(
    pl.kernel,
    mesh=plsc.VectorSubcoreMesh(core_axis_name="c", subcore_axis_name="s"),
    out_type=jax.ShapeDtypeStruct((ROWS * D,), jnp.float32),
    compiler_params=pltpu.CompilerParams(use_tc_tiling_on_sc=False),
    scratch_types=[
        pltpu.VMEM((ROWS_PER_W * PER_ROW,), jnp.int32),
        [pltpu.VMEM((IDX_BLK, D), jnp.float32) for _ in range(NBUF)],
        [pltpu.VMEM((R_BLK * D,), jnp.float32) for _ in range(NBUF)],
        [pltpu.SemaphoreType.DMA for _ in range(NBUF)],
        [pltpu.SemaphoreType.DMA for _ in range(NBUF)],
    ],
)
def _pooled_lookup(idx_hbm, table_hbm, out_hbm, idx_all, rows_bufs,
                   out_bufs, semg, semo):
    wid = lax.axis_index("s") * NC + lax.axis_index("c")
    row_base = wid * ROWS_PER_W

    pltpu.sync_copy(
        idx_hbm.at[pl.ds(row_base * PER_ROW, ROWS_PER_W * PER_ROW)], idx_all)

    def gather(i, rows_b, sem_b):
        return pltpu.make_async_copy(
            table_hbm.at[idx_all.at[pl.ds(i * IDX_BLK, IDX_BLK)]],
            rows_b, sem_b)

    def out_store(i, out_b, sem_b):
        return pltpu.make_async_copy(
            out_b, out_hbm.at[pl.ds((row_base + i * R_BLK) * D, R_BLK * D)],
            sem_b)

    for b in range(NBUF):
        gather(b, rows_bufs[b], semg[b]).start()

    zeros = jnp.zeros((LANES,), jnp.float32)

    def body(p, carry):
        for b in range(NBUF):
            rows_b, out_b, semg_b, semo_b = (
                rows_bufs[b], out_bufs[b], semg[b], semo[b])
            i = NBUF * p + b
            gather(i, rows_b, semg_b).wait()

            def jbody(jj, accs):
                accs = list(accs)
                for u in range(J_GRP):
                    for r in range(R_BLK):
                        row = r * PER_ROW + jj * J_GRP + u
                        for c in range(NCH):
                            accs[r * NCH + c] = accs[r * NCH + c] + (
                                rows_b[row, pl.ds(c * LANES, LANES)])
                return tuple(accs)

            accs = lax.fori_loop(0, PER_ROW // J_GRP, jbody,
                                 (zeros,) * (R_BLK * NCH))

            @pl.when(i + NBUF < N_BLK)
            def _():
                gather(i + NBUF, rows_b, semg_b).start()

            @pl.when(i >= NBUF)
            def _():
                out_store(i, out_b, semo_b).wait()

            for r in range(R_BLK):
                for c in range(NCH):
                    out_b[pl.ds(r * D + c * LANES, LANES)] = (
                        accs[r * NCH + c] * SCALE)
            out_store(i, out_b, semo_b).start()
        return carry

    lax.fori_loop(0, N_BLK // NBUF, body, 0)
    for b in range(NBUF):
        out_store(N_BLK - NBUF + b, out_bufs[b], semo[b]).wait()


def kernel(sub_index, derived_sub_indices, action_mask, table):
    obs_idx = sub_index.reshape(BATCH, PER_ROW)
    act_idx = derived_sub_indices.reshape(BATCH * STATES, PER_ROW)
    flat_idx = jnp.concatenate([obs_idx, act_idx], axis=0).reshape(-1)
    out = _pooled_lookup(flat_idx, table).reshape(ROWS, D)
    obs = out[:BATCH]
    act = out[BATCH:].reshape(BATCH, STATES, D)
    return (obs, act, action_mask)
